# baseline (speedup 1.0000x reference)
import functools
import math

import jax
import jax.numpy as jnp
from jax import lax
from jax.experimental import pallas as pl
from jax.experimental.pallas import tpu as pltpu
from jax.experimental.pallas import tpu_sc as plsc

S = 2048
D = 1024
H = 16
DK = 64
F = 2048
E = 8
V = 32000
SQRTD = math.sqrt(D)


def _sc_gather(table, idx):
    n_rows, d = table.shape
    b = idx.shape[0]
    dt = table.dtype
    itemsize = jnp.dtype(dt).itemsize
    info = plsc.get_sparse_core_info()
    nw = info.num_cores * info.num_subcores
    assert b % (8 * nw) == 0
    b_per_w = b // nw
    chunk = b_per_w
    while chunk * d * itemsize > 384 * 1024:
        chunk //= 2
    n_chunks = b_per_w // chunk
    mesh = plsc.VectorSubcoreMesh(core_axis_name="c", subcore_axis_name="s")

    @functools.partial(
        pl.kernel, mesh=mesh,
        out_type=jax.ShapeDtypeStruct((b, d), dt),
        scratch_types=[
            pltpu.VMEM((chunk,), jnp.int32),
            pltpu.VMEM((chunk, d), dt),
            pltpu.SemaphoreType.DMA,
        ],
    )
    def k(table_hbm, idx_hbm, out_hbm, idx_v, rows_v, sem):
        wid = lax.axis_index("s") * info.num_cores + lax.axis_index("c")
        base = wid * b_per_w
        for c in range(n_chunks):
            off = base + c * chunk
            pltpu.sync_copy(idx_hbm.at[pl.ds(off, chunk)], idx_v)
            pltpu.async_copy(table_hbm.at[idx_v], rows_v, sem).wait()
            pltpu.sync_copy(rows_v, out_hbm.at[pl.ds(off, chunk)])

    return k(table, idx)


def _sc_gather_bf16(table, idx):
    n_rows, d = table.shape
    t32 = jax.lax.bitcast_convert_type(
        table.reshape(n_rows, d // 2, 2), jnp.int32)
    out32 = _sc_gather(t32, idx)
    return jax.lax.bitcast_convert_type(
        out32, jnp.bfloat16).reshape(idx.shape[0], d)


def _rope_coefs():
    inv_freq = 1.0 / (10000.0 ** (jnp.arange(0, D, 2, dtype=jnp.float32) / D))
    t = jnp.arange(S, dtype=jnp.float32)
    si = t[:, None] * inv_freq[None, :]
    sin = jnp.sin(si)
    cos = jnp.cos(si)
    c = jnp.repeat(cos, 2, axis=1) * SQRTD
    dmask = (jnp.arange(D) % 2 == 0)
    a = jnp.where(dmask[None, :], -jnp.repeat(sin, 2, axis=1), 0.0) * SQRTD
    b = jnp.where(dmask[None, :], 0.0, jnp.repeat(sin, 2, axis=1)) * SQRTD
    return c, a, b


def _rope_kernel(x_ref, c_ref, a_ref, b_ref, o_ref):
    x = x_ref[...]
    shl = jnp.concatenate([x[:, 1:], x[:, :1]], axis=1)
    shr = jnp.concatenate([x[:, -1:], x[:, :-1]], axis=1)
    o_ref[...] = x * c_ref[...] + shl * a_ref[...] + shr * b_ref[...]


def _embed_rope(ids, emb, c, a, b):
    rows = _sc_gather(emb, ids)
    bs = 256
    spec = pl.BlockSpec((bs, D), lambda i: (i, 0))
    return pl.pallas_call(
        _rope_kernel,
        grid=(S // bs,),
        in_specs=[spec] * 4,
        out_specs=spec,
        out_shape=jax.ShapeDtypeStruct((S, D), jnp.float32),
    )(rows, c, a, b)


def _matmul_kernel(x_ref, w_ref, b_ref, o_ref):
    o_ref[...] = (jnp.dot(x_ref[...].astype(jnp.bfloat16), w_ref[...],
                          preferred_element_type=jnp.float32)
                  + b_ref[...])


def _matmul(x, w, b, bn):
    m, k = x.shape
    n = w.shape[1]
    assert n % bn == 0
    return pl.pallas_call(
        _matmul_kernel,
        grid=(n // bn,),
        in_specs=[
            pl.BlockSpec((m, k), lambda j: (0, 0)),
            pl.BlockSpec((k, bn), lambda j: (0, j)),
            pl.BlockSpec((1, bn), lambda j: (0, j)),
        ],
        out_specs=pl.BlockSpec((m, bn), lambda j: (0, j)),
        out_shape=jax.ShapeDtypeStruct((m, n), jnp.float32),
    )(x, w.astype(jnp.bfloat16), b.reshape(1, n))


_BQ = 256


def _qkv_proj_kernel(x_ref, w_ref, b_ref, o_ref):
    o_ref[0] = (jnp.dot(x_ref[...].astype(jnp.bfloat16), w_ref[0],
                        preferred_element_type=jnp.float32) + b_ref[0])


def _qkv_proj(x, w, b):
    w3 = w.reshape(D, H, DK).transpose(1, 0, 2).astype(jnp.bfloat16)
    b3 = b.reshape(H, 1, DK)
    return pl.pallas_call(
        _qkv_proj_kernel,
        grid=(H,),
        in_specs=[
            pl.BlockSpec((S, D), lambda h: (0, 0)),
            pl.BlockSpec((1, D, DK), lambda h: (h, 0, 0)),
            pl.BlockSpec((1, 1, DK), lambda h: (h, 0, 0)),
        ],
        out_specs=pl.BlockSpec((1, S, DK), lambda h: (h, 0, 0)),
        out_shape=jax.ShapeDtypeStruct((H, S, DK), jnp.float32),
    )(x, w3, b3)


def _attn_kernel(q_ref, k_ref, v_ref, o_ref):
    s = jax.lax.dot_general(q_ref[0].astype(jnp.bfloat16),
                            k_ref[0].astype(jnp.bfloat16),
                            (((1,), (1,)), ((), ())),
                            preferred_element_type=jnp.float32)
    s = s * (1.0 / math.sqrt(DK))
    mx = jnp.max(s, axis=-1, keepdims=True)
    p = jnp.exp(s - mx)
    o = jnp.dot(p.astype(jnp.bfloat16), v_ref[0].astype(jnp.bfloat16),
                preferred_element_type=jnp.float32)
    o_ref[0] = o / jnp.sum(p, axis=-1, keepdims=True)


def _attention(q, k, v):
    return pl.pallas_call(
        _attn_kernel,
        grid=(H, S // _BQ),
        in_specs=[
            pl.BlockSpec((1, _BQ, DK), lambda h, i: (h, i, 0)),
            pl.BlockSpec((1, S, DK), lambda h, i: (h, 0, 0)),
            pl.BlockSpec((1, S, DK), lambda h, i: (h, 0, 0)),
        ],
        out_specs=pl.BlockSpec((1, _BQ, DK), lambda h, i: (h, i, 0)),
        out_shape=jax.ShapeDtypeStruct((H, S, DK), jnp.float32),
    )(q, k, v)


def _o_proj_kernel(a_ref, w_ref, b_ref, o_ref):
    h = pl.program_id(0)
    part = jnp.dot(a_ref[0].astype(jnp.bfloat16), w_ref[0],
                   preferred_element_type=jnp.float32)

    @pl.when(h == 0)
    def _init():
        o_ref[...] = part + b_ref[...]

    @pl.when(h != 0)
    def _acc():
        o_ref[...] += part


def _o_proj(a, w, b):
    w3 = w.reshape(H, DK, D).astype(jnp.bfloat16)
    return pl.pallas_call(
        _o_proj_kernel,
        grid=(H,),
        in_specs=[
            pl.BlockSpec((1, S, DK), lambda h: (h, 0, 0)),
            pl.BlockSpec((1, DK, D), lambda h: (h, 0, 0)),
            pl.BlockSpec((1, D), lambda h: (0, 0)),
        ],
        out_specs=pl.BlockSpec((S, D), lambda h: (0, 0)),
        out_shape=jax.ShapeDtypeStruct((S, D), jnp.float32),
    )(a, w3, b.reshape(1, D))


def _mha(p, q, k, v):
    Q = _qkv_proj(q, p['Wq'], p['bq'])
    K = _qkv_proj(k, p['Wk'], p['bk'])
    Vv = _qkv_proj(v, p['Wv'], p['bv'])
    o = _attention(Q, K, Vv)
    return _o_proj(o, p['Wo'], p['bo'])


def _add_ln_kernel(x_ref, d_ref, g_ref, b_ref, o_ref, o16_ref):
    y = x_ref[...] + d_ref[...]
    mu = jnp.mean(y, axis=-1, keepdims=True)
    yc = y - mu
    var = jnp.mean(yc * yc, axis=-1, keepdims=True)
    out = yc * jax.lax.rsqrt(var + 1e-5) * g_ref[...] + b_ref[...]
    o_ref[...] = out
    o16_ref[...] = out.astype(jnp.bfloat16)


def _add_ln(x, delta, lnp):
    bs = 256
    spec = pl.BlockSpec((bs, D), lambda i: (i, 0))
    return pl.pallas_call(
        _add_ln_kernel,
        grid=(S // bs,),
        in_specs=[
            spec,
            spec,
            pl.BlockSpec((1, D), lambda i: (0, 0)),
            pl.BlockSpec((1, D), lambda i: (0, 0)),
        ],
        out_specs=(spec, spec),
        out_shape=(jax.ShapeDtypeStruct((S, D), jnp.float32),
                   jax.ShapeDtypeStruct((S, D), jnp.bfloat16)),
    )(x, delta, lnp['g'].reshape(1, D), lnp['b'].reshape(1, D))


def _gate_kernel(x_ref, w_ref, b_ref, meta_ref):
    s = jnp.dot(x_ref[...], w_ref[...],
                preferred_element_type=jnp.float32) + b_ref[...]
    cols = jax.lax.broadcasted_iota(jnp.int32, s.shape, 1)
    m1 = jnp.max(s, axis=-1, keepdims=True)
    i1 = jnp.min(jnp.where(s == m1, cols, E), axis=-1, keepdims=True)
    s2 = jnp.where(cols == i1, -jnp.inf, s)
    m2 = jnp.max(s2, axis=-1, keepdims=True)
    i2 = jnp.min(jnp.where(s2 == m2, cols, E), axis=-1, keepdims=True)
    ex = jnp.exp(m2 - m1)
    w1 = 1.0 / (1.0 + ex)
    w2 = 1.0 - w1
    meta_ref[...] = jnp.concatenate(
        [w1, w2, i1.astype(jnp.float32), i2.astype(jnp.float32),
         jnp.zeros_like(s[:, :4])], axis=1)


def _gate(x, gw, gb):
    bs = 256
    return pl.pallas_call(
        _gate_kernel,
        grid=(S // bs,),
        in_specs=[
            pl.BlockSpec((bs, D), lambda i: (i, 0)),
            pl.BlockSpec((D, E), lambda i: (0, 0)),
            pl.BlockSpec((1, E), lambda i: (0, 0)),
        ],
        out_specs=pl.BlockSpec((bs, E), lambda i: (i, 0)),
        out_shape=jax.ShapeDtypeStruct((S, E), jnp.float32),
    )(x, gw, gb.reshape(1, E))


_BM = 128
_PT = 2 * S + E * _BM
_NMB = _PT // _BM

def _expert_kernel(be_ref, xs_ref, ws_ref, w1_ref, b1_ref, w2_ref, b2_ref,
                   o_ref):
    e = be_ref[pl.program_id(0)]
    h = (jnp.dot(xs_ref[...], w1_ref[0],
                 preferred_element_type=jnp.float32) + b1_ref[0])
    gelu = 0.5 * h * (1.0 + jax.lax.erf(h * (1.0 / math.sqrt(2.0))))
    silu = h * jax.nn.sigmoid(h)
    h = jnp.where(e % 2 == 0, gelu, silu)
    part = jnp.dot(h.astype(jnp.bfloat16), w2_ref[0],
                   preferred_element_type=jnp.float32)
    o_ref[...] = (ws_ref[...] * (part + b2_ref[0])).astype(jnp.bfloat16)


def _expert_ffn(xs, ws, be, W1, b1, W2, b2):
    spec = pltpu.PrefetchScalarGridSpec(
        num_scalar_prefetch=1,
        grid=(_NMB,),
        in_specs=[
            pl.BlockSpec((_BM, D), lambda mb, be: (mb, 0)),
            pl.BlockSpec((_BM, 1), lambda mb, be: (mb, 0)),
            pl.BlockSpec((1, D, F), lambda mb, be: (be[mb], 0, 0)),
            pl.BlockSpec((1, 1, F), lambda mb, be: (be[mb], 0, 0)),
            pl.BlockSpec((1, F, D), lambda mb, be: (be[mb], 0, 0)),
            pl.BlockSpec((1, 1, D), lambda mb, be: (be[mb], 0, 0)),
        ],
        out_specs=pl.BlockSpec((_BM, D), lambda mb, be: (mb, 0)),
    )
    return pl.pallas_call(
        _expert_kernel,
        grid_spec=spec,
        out_shape=jax.ShapeDtypeStruct((_PT, D), jnp.bfloat16),
    )(be, xs, ws, W1.astype(jnp.bfloat16), b1.reshape(E, 1, F),
      W2.astype(jnp.bfloat16), b2.reshape(E, 1, D))


def _combine_ln_kernel(ya_ref, yb_ref, x_ref, g_ref, b_ref, o_ref):
    y = (ya_ref[...].astype(jnp.float32) + yb_ref[...].astype(jnp.float32)
         + x_ref[...])
    mu = jnp.mean(y, axis=-1, keepdims=True)
    yc = y - mu
    var = jnp.mean(yc * yc, axis=-1, keepdims=True)
    o_ref[...] = yc * jax.lax.rsqrt(var + 1e-5) * g_ref[...] + b_ref[...]


def _combine_ln(ypair, x, lnp):
    bs = 256
    spec = pl.BlockSpec((bs, D), lambda i: (i, 0))
    return pl.pallas_call(
        _combine_ln_kernel,
        grid=(S // bs,),
        in_specs=[
            spec,
            pl.BlockSpec((bs, D), lambda i: (i + S // bs, 0)),
            spec,
            pl.BlockSpec((1, D), lambda i: (0, 0)),
            pl.BlockSpec((1, D), lambda i: (0, 0)),
        ],
        out_specs=spec,
        out_shape=jax.ShapeDtypeStruct((S, D), jnp.float32),
    )(ypair, ypair, x, lnp['g'].reshape(1, D), lnp['b'].reshape(1, D))


def _moe_ln(p, x, xb, lnp):
    meta = _gate(x, p['gate_W'], p['gate_b'])
    w1, w2 = meta[:, 0], meta[:, 1]
    e1 = meta[:, 2].astype(jnp.int32)
    e2 = meta[:, 3].astype(jnp.int32)

    ef = jnp.concatenate([e1, e2])
    wf = jnp.concatenate([w1, w2])
    tf = jnp.concatenate([jnp.arange(S, dtype=jnp.int32)] * 2)
    order = jnp.argsort(ef)
    ef_s, tf_s, wf_s = ef[order], tf[order], wf[order]
    counts = jnp.bincount(ef, length=E)
    starts = jnp.cumsum(counts) - counts
    pcounts = ((counts + _BM - 1) // _BM) * _BM
    pcsum = jnp.cumsum(pcounts)
    pstarts = pcsum - pcounts
    rank = jnp.arange(2 * S, dtype=jnp.int32) - starts[ef_s]
    pos_s = (pstarts[ef_s] + rank).astype(jnp.int32)
    src_row = jnp.zeros((_PT,), jnp.int32).at[pos_s].set(tf_s)
    ws = jnp.zeros((_PT,), jnp.float32).at[pos_s].set(wf_s)
    mbs = jnp.arange(_NMB, dtype=jnp.int32) * _BM
    be = jnp.clip(jnp.sum(pcsum[None, :] <= mbs[:, None], axis=1),
                  0, E - 1).astype(jnp.int32)
    posf = jnp.zeros((2 * S,), jnp.int32).at[order].set(pos_s)
    p1, p2 = posf[:S], posf[S:]

    xs = _sc_gather_bf16(xb, src_row)
    ys = _expert_ffn(xs, ws.reshape(_PT, 1), be, p['W1'], p['b1'],
                     p['W2'], p['b2'])
    ypair = _sc_gather_bf16(ys, jnp.concatenate([p1, p2]))
    return _combine_ln(ypair, x, lnp)


def _block(p, x, enc_out=None):
    x, xb = _add_ln(x, _mha(p['sa'], x, x, x), p['ln1'])
    if enc_out is not None:
        x, xb = _add_ln(x, _mha(p['ca'], x, enc_out, enc_out), p['ln2'])
    return _moe_ln(p['moe'], x, xb, p['ln3'])


def kernel(src, tgt, params):
    src = src.reshape(-1).astype(jnp.int32)
    tgt = tgt.reshape(-1).astype(jnp.int32)
    emb = params['embedding']
    c, a, b = _rope_coefs()
    se = _embed_rope(src, emb, c, a, b)
    se = _block(params['enc'][0], se)
    te = _embed_rope(tgt, emb, c, a, b)
    te = _block(params['dec'][0], te, enc_out=se)
    logits = _matmul(te, params['out_W'], params['out_b'], 640)
    return logits.reshape(1, S, -1)


# speedup vs baseline: 1.3689x; 1.3689x over previous
import functools
import math

import jax
import jax.numpy as jnp
from jax import lax
from jax.experimental import pallas as pl
from jax.experimental.pallas import tpu as pltpu
from jax.experimental.pallas import tpu_sc as plsc

S = 2048
D = 1024
H = 16
DK = 64
F = 2048
E = 8
V = 32000
SQRTD = math.sqrt(D)


def _sc_gather(table, idx):
    n_rows, d = table.shape
    b = idx.shape[0]
    dt = table.dtype
    itemsize = jnp.dtype(dt).itemsize
    info = plsc.get_sparse_core_info()
    nw = info.num_cores * info.num_subcores
    assert b % (8 * nw) == 0
    b_per_w = b // nw
    chunk = b_per_w
    while chunk * d * itemsize > 384 * 1024:
        chunk //= 2
    n_chunks = b_per_w // chunk
    mesh = plsc.VectorSubcoreMesh(core_axis_name="c", subcore_axis_name="s")

    @functools.partial(
        pl.kernel, mesh=mesh,
        out_type=jax.ShapeDtypeStruct((b, d), dt),
        scratch_types=[
            pltpu.VMEM((chunk,), jnp.int32),
            pltpu.VMEM((chunk, d), dt),
            pltpu.SemaphoreType.DMA,
        ],
    )
    def k(table_hbm, idx_hbm, out_hbm, idx_v, rows_v, sem):
        wid = lax.axis_index("s") * info.num_cores + lax.axis_index("c")
        base = wid * b_per_w
        for c in range(n_chunks):
            off = base + c * chunk
            pltpu.sync_copy(idx_hbm.at[pl.ds(off, chunk)], idx_v)
            pltpu.async_copy(table_hbm.at[idx_v], rows_v, sem).wait()
            pltpu.sync_copy(rows_v, out_hbm.at[pl.ds(off, chunk)])

    return k(table, idx)


def _rope_coefs():
    inv_freq = 1.0 / (10000.0 ** (jnp.arange(0, D, 2, dtype=jnp.float32) / D))
    t = jnp.arange(S, dtype=jnp.float32)
    si = t[:, None] * inv_freq[None, :]
    sin = jnp.sin(si)
    cos = jnp.cos(si)
    c = jnp.repeat(cos, 2, axis=1) * SQRTD
    dmask = (jnp.arange(D) % 2 == 0)
    a = jnp.where(dmask[None, :], -jnp.repeat(sin, 2, axis=1), 0.0) * SQRTD
    b = jnp.where(dmask[None, :], 0.0, jnp.repeat(sin, 2, axis=1)) * SQRTD
    return c, a, b


def _rope_kernel(x_ref, c_ref, a_ref, b_ref, o_ref):
    x = x_ref[...]
    shl = jnp.concatenate([x[:, 1:], x[:, :1]], axis=1)
    shr = jnp.concatenate([x[:, -1:], x[:, :-1]], axis=1)
    o_ref[...] = x * c_ref[...] + shl * a_ref[...] + shr * b_ref[...]


def _embed_rope(ids, emb, c, a, b):
    rows = _sc_gather(emb, ids)
    bs = 256
    spec = pl.BlockSpec((bs, D), lambda i: (i, 0))
    return pl.pallas_call(
        _rope_kernel,
        grid=(S // bs,),
        in_specs=[spec] * 4,
        out_specs=spec,
        out_shape=jax.ShapeDtypeStruct((S, D), jnp.float32),
    )(rows, c, a, b)


def _matmul_kernel(x_ref, w_ref, b_ref, o_ref):
    o_ref[...] = (jnp.dot(x_ref[...], w_ref[...],
                          preferred_element_type=jnp.float32)
                  + b_ref[...])


def _matmul(x, w, b, bn):
    m, k = x.shape
    n = w.shape[1]
    assert n % bn == 0
    return pl.pallas_call(
        _matmul_kernel,
        grid=(n // bn,),
        in_specs=[
            pl.BlockSpec((m, k), lambda j: (0, 0)),
            pl.BlockSpec((k, bn), lambda j: (0, j)),
            pl.BlockSpec((1, bn), lambda j: (0, j)),
        ],
        out_specs=pl.BlockSpec((m, bn), lambda j: (0, j)),
        out_shape=jax.ShapeDtypeStruct((m, n), jnp.float32),
    )(x, w, b.reshape(1, n))


_BQ = 256


def _qkv_proj_kernel(x_ref, w_ref, b_ref, o_ref):
    o_ref[0] = (jnp.dot(x_ref[...], w_ref[0],
                        preferred_element_type=jnp.float32) + b_ref[0])


def _qkv_proj(x, w, b):
    w3 = w.reshape(D, H, DK).transpose(1, 0, 2)
    b3 = b.reshape(H, 1, DK)
    return pl.pallas_call(
        _qkv_proj_kernel,
        grid=(H,),
        in_specs=[
            pl.BlockSpec((S, D), lambda h: (0, 0)),
            pl.BlockSpec((1, D, DK), lambda h: (h, 0, 0)),
            pl.BlockSpec((1, 1, DK), lambda h: (h, 0, 0)),
        ],
        out_specs=pl.BlockSpec((1, S, DK), lambda h: (h, 0, 0)),
        out_shape=jax.ShapeDtypeStruct((H, S, DK), jnp.float32),
    )(x, w3, b3)


def _attn_kernel(q_ref, k_ref, v_ref, o_ref):
    s = jax.lax.dot_general(q_ref[0], k_ref[0],
                            (((1,), (1,)), ((), ())),
                            preferred_element_type=jnp.float32)
    s = s * (1.0 / math.sqrt(DK))
    mx = jnp.max(s, axis=-1, keepdims=True)
    p = jnp.exp(s - mx)
    o = jnp.dot(p, v_ref[0], preferred_element_type=jnp.float32)
    o_ref[0] = o / jnp.sum(p, axis=-1, keepdims=True)


def _attention(q, k, v):
    return pl.pallas_call(
        _attn_kernel,
        grid=(H, S // _BQ),
        in_specs=[
            pl.BlockSpec((1, _BQ, DK), lambda h, i: (h, i, 0)),
            pl.BlockSpec((1, S, DK), lambda h, i: (h, 0, 0)),
            pl.BlockSpec((1, S, DK), lambda h, i: (h, 0, 0)),
        ],
        out_specs=pl.BlockSpec((1, _BQ, DK), lambda h, i: (h, i, 0)),
        out_shape=jax.ShapeDtypeStruct((H, S, DK), jnp.float32),
    )(q, k, v)


def _o_proj_kernel(a_ref, w_ref, b_ref, o_ref):
    h = pl.program_id(0)
    part = jnp.dot(a_ref[0], w_ref[0], preferred_element_type=jnp.float32)

    @pl.when(h == 0)
    def _init():
        o_ref[...] = part + b_ref[...]

    @pl.when(h != 0)
    def _acc():
        o_ref[...] += part


def _o_proj(a, w, b):
    w3 = w.reshape(H, DK, D)
    return pl.pallas_call(
        _o_proj_kernel,
        grid=(H,),
        in_specs=[
            pl.BlockSpec((1, S, DK), lambda h: (h, 0, 0)),
            pl.BlockSpec((1, DK, D), lambda h: (h, 0, 0)),
            pl.BlockSpec((1, D), lambda h: (0, 0)),
        ],
        out_specs=pl.BlockSpec((S, D), lambda h: (0, 0)),
        out_shape=jax.ShapeDtypeStruct((S, D), jnp.float32),
    )(a, w3, b.reshape(1, D))


def _mha(p, q, k, v):
    Q = _qkv_proj(q, p['Wq'], p['bq'])
    K = _qkv_proj(k, p['Wk'], p['bk'])
    Vv = _qkv_proj(v, p['Wv'], p['bv'])
    o = _attention(Q, K, Vv)
    return _o_proj(o, p['Wo'], p['bo'])


def _add_ln_kernel(x_ref, d_ref, g_ref, b_ref, o_ref):
    y = x_ref[...] + d_ref[...]
    mu = jnp.mean(y, axis=-1, keepdims=True)
    yc = y - mu
    var = jnp.mean(yc * yc, axis=-1, keepdims=True)
    o_ref[...] = yc * jax.lax.rsqrt(var + 1e-5) * g_ref[...] + b_ref[...]


def _add_ln(x, delta, lnp):
    bs = 256
    spec = pl.BlockSpec((bs, D), lambda i: (i, 0))
    return pl.pallas_call(
        _add_ln_kernel,
        grid=(S // bs,),
        in_specs=[
            spec,
            spec,
            pl.BlockSpec((1, D), lambda i: (0, 0)),
            pl.BlockSpec((1, D), lambda i: (0, 0)),
        ],
        out_specs=spec,
        out_shape=jax.ShapeDtypeStruct((S, D), jnp.float32),
    )(x, delta, lnp['g'].reshape(1, D), lnp['b'].reshape(1, D))


def _gate_kernel(x_ref, w_ref, b_ref, meta_ref):
    s = jnp.dot(x_ref[...], w_ref[...],
                preferred_element_type=jnp.float32) + b_ref[...]
    cols = jax.lax.broadcasted_iota(jnp.int32, s.shape, 1)
    m1 = jnp.max(s, axis=-1, keepdims=True)
    i1 = jnp.min(jnp.where(s == m1, cols, E), axis=-1, keepdims=True)
    s2 = jnp.where(cols == i1, -jnp.inf, s)
    m2 = jnp.max(s2, axis=-1, keepdims=True)
    i2 = jnp.min(jnp.where(s2 == m2, cols, E), axis=-1, keepdims=True)
    ex = jnp.exp(m2 - m1)
    w1 = 1.0 / (1.0 + ex)
    w2 = 1.0 - w1
    meta_ref[...] = jnp.concatenate(
        [w1, w2, i1.astype(jnp.float32), i2.astype(jnp.float32),
         jnp.zeros_like(s[:, :4])], axis=1)


def _gate(x, gw, gb):
    bs = 256
    return pl.pallas_call(
        _gate_kernel,
        grid=(S // bs,),
        in_specs=[
            pl.BlockSpec((bs, D), lambda i: (i, 0)),
            pl.BlockSpec((D, E), lambda i: (0, 0)),
            pl.BlockSpec((1, E), lambda i: (0, 0)),
        ],
        out_specs=pl.BlockSpec((bs, E), lambda i: (i, 0)),
        out_shape=jax.ShapeDtypeStruct((S, E), jnp.float32),
    )(x, gw, gb.reshape(1, E))


_BM = 128
_PT = 2 * S + E * _BM
_NMB = _PT // _BM

def _expert_kernel(be_ref, xs_ref, ws_ref, w1_ref, b1_ref, w2_ref, b2_ref,
                   o_ref):
    e = be_ref[pl.program_id(0)]
    h = (jnp.dot(xs_ref[...], w1_ref[0],
                 preferred_element_type=jnp.float32) + b1_ref[0])
    gelu = 0.5 * h * (1.0 + jax.lax.erf(h * (1.0 / math.sqrt(2.0))))
    silu = h * jax.nn.sigmoid(h)
    h = jnp.where(e % 2 == 0, gelu, silu)
    part = jnp.dot(h, w2_ref[0], preferred_element_type=jnp.float32)
    o_ref[...] = ws_ref[...] * (part + b2_ref[0])


def _expert_ffn(xs, ws, be, W1, b1, W2, b2):
    spec = pltpu.PrefetchScalarGridSpec(
        num_scalar_prefetch=1,
        grid=(_NMB,),
        in_specs=[
            pl.BlockSpec((_BM, D), lambda mb, be: (mb, 0)),
            pl.BlockSpec((_BM, 1), lambda mb, be: (mb, 0)),
            pl.BlockSpec((1, D, F), lambda mb, be: (be[mb], 0, 0)),
            pl.BlockSpec((1, 1, F), lambda mb, be: (be[mb], 0, 0)),
            pl.BlockSpec((1, F, D), lambda mb, be: (be[mb], 0, 0)),
            pl.BlockSpec((1, 1, D), lambda mb, be: (be[mb], 0, 0)),
        ],
        out_specs=pl.BlockSpec((_BM, D), lambda mb, be: (mb, 0)),
    )
    return pl.pallas_call(
        _expert_kernel,
        grid_spec=spec,
        out_shape=jax.ShapeDtypeStruct((_PT, D), jnp.float32),
    )(be, xs, ws, W1, b1.reshape(E, 1, F), W2, b2.reshape(E, 1, D))


def _combine_ln_kernel(ya_ref, yb_ref, x_ref, g_ref, b_ref, o_ref):
    y = ya_ref[...] + yb_ref[...] + x_ref[...]
    mu = jnp.mean(y, axis=-1, keepdims=True)
    yc = y - mu
    var = jnp.mean(yc * yc, axis=-1, keepdims=True)
    o_ref[...] = yc * jax.lax.rsqrt(var + 1e-5) * g_ref[...] + b_ref[...]


def _combine_ln(ypair, x, lnp):
    bs = 256
    spec = pl.BlockSpec((bs, D), lambda i: (i, 0))
    return pl.pallas_call(
        _combine_ln_kernel,
        grid=(S // bs,),
        in_specs=[
            spec,
            pl.BlockSpec((bs, D), lambda i: (i + S // bs, 0)),
            spec,
            pl.BlockSpec((1, D), lambda i: (0, 0)),
            pl.BlockSpec((1, D), lambda i: (0, 0)),
        ],
        out_specs=spec,
        out_shape=jax.ShapeDtypeStruct((S, D), jnp.float32),
    )(ypair, ypair, x, lnp['g'].reshape(1, D), lnp['b'].reshape(1, D))


def _moe_ln(p, x, lnp):
    meta = _gate(x, p['gate_W'], p['gate_b'])
    w1, w2 = meta[:, 0], meta[:, 1]
    e1 = meta[:, 2].astype(jnp.int32)
    e2 = meta[:, 3].astype(jnp.int32)

    ef = jnp.concatenate([e1, e2])
    wf = jnp.concatenate([w1, w2])
    tf = jnp.concatenate([jnp.arange(S, dtype=jnp.int32)] * 2)
    order = jnp.argsort(ef)
    ef_s, tf_s, wf_s = ef[order], tf[order], wf[order]
    counts = jnp.bincount(ef, length=E)
    starts = jnp.cumsum(counts) - counts
    pcounts = ((counts + _BM - 1) // _BM) * _BM
    pcsum = jnp.cumsum(pcounts)
    pstarts = pcsum - pcounts
    rank = jnp.arange(2 * S, dtype=jnp.int32) - starts[ef_s]
    pos_s = (pstarts[ef_s] + rank).astype(jnp.int32)
    src_row = jnp.zeros((_PT,), jnp.int32).at[pos_s].set(tf_s)
    ws = jnp.zeros((_PT,), jnp.float32).at[pos_s].set(wf_s)
    mbs = jnp.arange(_NMB, dtype=jnp.int32) * _BM
    be = jnp.clip(jnp.sum(pcsum[None, :] <= mbs[:, None], axis=1),
                  0, E - 1).astype(jnp.int32)
    posf = jnp.zeros((2 * S,), jnp.int32).at[order].set(pos_s)
    p1, p2 = posf[:S], posf[S:]

    xs = _sc_gather(x, src_row)
    ys = _expert_ffn(xs, ws.reshape(_PT, 1), be, p['W1'], p['b1'],
                     p['W2'], p['b2'])
    ypair = _sc_gather(ys, jnp.concatenate([p1, p2]))
    return _combine_ln(ypair, x, lnp)


def _block(p, x, enc_out=None):
    x = _add_ln(x, _mha(p['sa'], x, x, x), p['ln1'])
    if enc_out is not None:
        x = _add_ln(x, _mha(p['ca'], x, enc_out, enc_out), p['ln2'])
    return _moe_ln(p['moe'], x, p['ln3'])


def kernel(src, tgt, params):
    src = src.reshape(-1).astype(jnp.int32)
    tgt = tgt.reshape(-1).astype(jnp.int32)
    emb = params['embedding']
    c, a, b = _rope_coefs()
    se = _embed_rope(src, emb, c, a, b)
    se = _block(params['enc'][0], se)
    te = _embed_rope(tgt, emb, c, a, b)
    te = _block(params['dec'][0], te, enc_out=se)
    logits = _matmul(te, params['out_W'], params['out_b'], 640)
    return logits.reshape(1, S, -1)


# speedup vs baseline: 1.3733x; 1.0032x over previous
import functools
import math

import jax
import jax.numpy as jnp
from jax import lax
from jax.experimental import pallas as pl
from jax.experimental.pallas import tpu as pltpu
from jax.experimental.pallas import tpu_sc as plsc

S = 2048
D = 1024
H = 16
DK = 64
F = 2048
E = 8
V = 32000
SQRTD = math.sqrt(D)


def _sc_gather(table, idx):
    n_rows, d = table.shape
    b = idx.shape[0]
    dt = table.dtype
    itemsize = jnp.dtype(dt).itemsize
    info = plsc.get_sparse_core_info()
    nw = info.num_cores * info.num_subcores
    assert b % (8 * nw) == 0
    b_per_w = b // nw
    chunk = b_per_w
    while chunk * d * itemsize > 384 * 1024:
        chunk //= 2
    n_chunks = b_per_w // chunk
    mesh = plsc.VectorSubcoreMesh(core_axis_name="c", subcore_axis_name="s")

    @functools.partial(
        pl.kernel, mesh=mesh,
        out_type=jax.ShapeDtypeStruct((b, d), dt),
        scratch_types=[
            pltpu.VMEM((chunk,), jnp.int32),
            pltpu.VMEM((chunk, d), dt),
            pltpu.SemaphoreType.DMA,
        ],
    )
    def k(table_hbm, idx_hbm, out_hbm, idx_v, rows_v, sem):
        wid = lax.axis_index("s") * info.num_cores + lax.axis_index("c")
        base = wid * b_per_w
        for c in range(n_chunks):
            off = base + c * chunk
            pltpu.sync_copy(idx_hbm.at[pl.ds(off, chunk)], idx_v)
            pltpu.async_copy(table_hbm.at[idx_v], rows_v, sem).wait()
            pltpu.sync_copy(rows_v, out_hbm.at[pl.ds(off, chunk)])

    return k(table, idx)


def _rope_coefs():
    inv_freq = 1.0 / (10000.0 ** (jnp.arange(0, D, 2, dtype=jnp.float32) / D))
    t = jnp.arange(S, dtype=jnp.float32)
    si = t[:, None] * inv_freq[None, :]
    sin = jnp.sin(si)
    cos = jnp.cos(si)
    c = jnp.repeat(cos, 2, axis=1) * SQRTD
    dmask = (jnp.arange(D) % 2 == 0)
    a = jnp.where(dmask[None, :], -jnp.repeat(sin, 2, axis=1), 0.0) * SQRTD
    b = jnp.where(dmask[None, :], 0.0, jnp.repeat(sin, 2, axis=1)) * SQRTD
    return c, a, b


def _rope_kernel(x_ref, c_ref, a_ref, b_ref, o_ref):
    x = x_ref[...]
    shl = jnp.concatenate([x[:, 1:], x[:, :1]], axis=1)
    shr = jnp.concatenate([x[:, -1:], x[:, :-1]], axis=1)
    o_ref[...] = x * c_ref[...] + shl * a_ref[...] + shr * b_ref[...]


def _embed_rope2(src_ids, tgt_ids, emb, c, a, b):
    rows = _sc_gather(emb, jnp.concatenate([src_ids, tgt_ids]))
    bs = 256

    def rope_half(off):
        spec = pl.BlockSpec((bs, D), lambda i: (i, 0))
        return pl.pallas_call(
            _rope_kernel,
            grid=(S // bs,),
            in_specs=[pl.BlockSpec((bs, D),
                                   functools.partial(
                                       lambda i, o: (i + o, 0), o=off))]
            + [spec] * 3,
            out_specs=spec,
            out_shape=jax.ShapeDtypeStruct((S, D), jnp.float32),
        )(rows, c, a, b)

    return rope_half(0), rope_half(S // bs)


def _matmul_kernel(x_ref, w_ref, b_ref, o_ref):
    o_ref[...] = (jnp.dot(x_ref[...], w_ref[...],
                          preferred_element_type=jnp.float32)
                  + b_ref[...])


def _matmul(x, w, b, bn):
    m, k = x.shape
    n = w.shape[1]
    assert n % bn == 0
    return pl.pallas_call(
        _matmul_kernel,
        grid=(n // bn,),
        in_specs=[
            pl.BlockSpec((m, k), lambda j: (0, 0)),
            pl.BlockSpec((k, bn), lambda j: (0, j)),
            pl.BlockSpec((1, bn), lambda j: (0, j)),
        ],
        out_specs=pl.BlockSpec((m, bn), lambda j: (0, j)),
        out_shape=jax.ShapeDtypeStruct((m, n), jnp.float32),
    )(x, w, b.reshape(1, n))


_BQ = 256


def _qkv_proj_kernel(x_ref, w_ref, b_ref, o_ref):
    o_ref[0] = (jnp.dot(x_ref[...], w_ref[0],
                        preferred_element_type=jnp.float32) + b_ref[0])


def _qkv_proj(x, w, b):
    w3 = w.reshape(D, H, DK).transpose(1, 0, 2)
    b3 = b.reshape(H, 1, DK)
    return pl.pallas_call(
        _qkv_proj_kernel,
        grid=(H,),
        in_specs=[
            pl.BlockSpec((S, D), lambda h: (0, 0)),
            pl.BlockSpec((1, D, DK), lambda h: (h, 0, 0)),
            pl.BlockSpec((1, 1, DK), lambda h: (h, 0, 0)),
        ],
        out_specs=pl.BlockSpec((1, S, DK), lambda h: (h, 0, 0)),
        out_shape=jax.ShapeDtypeStruct((H, S, DK), jnp.float32),
    )(x, w3, b3)


def _attn_kernel(q_ref, k_ref, v_ref, o_ref):
    s = jax.lax.dot_general(q_ref[0], k_ref[0],
                            (((1,), (1,)), ((), ())),
                            preferred_element_type=jnp.float32)
    s = s * (1.0 / math.sqrt(DK))
    mx = jnp.max(s, axis=-1, keepdims=True)
    p = jnp.exp(s - mx)
    o = jnp.dot(p, v_ref[0], preferred_element_type=jnp.float32)
    o_ref[0] = o / jnp.sum(p, axis=-1, keepdims=True)


def _attention(q, k, v):
    return pl.pallas_call(
        _attn_kernel,
        grid=(H, S // _BQ),
        in_specs=[
            pl.BlockSpec((1, _BQ, DK), lambda h, i: (h, i, 0)),
            pl.BlockSpec((1, S, DK), lambda h, i: (h, 0, 0)),
            pl.BlockSpec((1, S, DK), lambda h, i: (h, 0, 0)),
        ],
        out_specs=pl.BlockSpec((1, _BQ, DK), lambda h, i: (h, i, 0)),
        out_shape=jax.ShapeDtypeStruct((H, S, DK), jnp.float32),
    )(q, k, v)


def _o_proj_kernel(a_ref, w_ref, b_ref, o_ref):
    h = pl.program_id(0)
    part = jnp.dot(a_ref[0], w_ref[0], preferred_element_type=jnp.float32)

    @pl.when(h == 0)
    def _init():
        o_ref[...] = part + b_ref[...]

    @pl.when(h != 0)
    def _acc():
        o_ref[...] += part


def _o_proj(a, w, b):
    w3 = w.reshape(H, DK, D)
    return pl.pallas_call(
        _o_proj_kernel,
        grid=(H,),
        in_specs=[
            pl.BlockSpec((1, S, DK), lambda h: (h, 0, 0)),
            pl.BlockSpec((1, DK, D), lambda h: (h, 0, 0)),
            pl.BlockSpec((1, D), lambda h: (0, 0)),
        ],
        out_specs=pl.BlockSpec((S, D), lambda h: (0, 0)),
        out_shape=jax.ShapeDtypeStruct((S, D), jnp.float32),
    )(a, w3, b.reshape(1, D))


def _mha(p, q, k, v):
    Q = _qkv_proj(q, p['Wq'], p['bq'])
    K = _qkv_proj(k, p['Wk'], p['bk'])
    Vv = _qkv_proj(v, p['Wv'], p['bv'])
    o = _attention(Q, K, Vv)
    return _o_proj(o, p['Wo'], p['bo'])


def _add_ln_kernel(x_ref, d_ref, g_ref, b_ref, o_ref):
    y = x_ref[...] + d_ref[...]
    mu = jnp.mean(y, axis=-1, keepdims=True)
    yc = y - mu
    var = jnp.mean(yc * yc, axis=-1, keepdims=True)
    o_ref[...] = yc * jax.lax.rsqrt(var + 1e-5) * g_ref[...] + b_ref[...]


def _add_ln(x, delta, lnp):
    bs = 256
    spec = pl.BlockSpec((bs, D), lambda i: (i, 0))
    return pl.pallas_call(
        _add_ln_kernel,
        grid=(S // bs,),
        in_specs=[
            spec,
            spec,
            pl.BlockSpec((1, D), lambda i: (0, 0)),
            pl.BlockSpec((1, D), lambda i: (0, 0)),
        ],
        out_specs=spec,
        out_shape=jax.ShapeDtypeStruct((S, D), jnp.float32),
    )(x, delta, lnp['g'].reshape(1, D), lnp['b'].reshape(1, D))


def _gate_kernel(x_ref, w_ref, b_ref, meta_ref):
    s = jnp.dot(x_ref[...], w_ref[...],
                preferred_element_type=jnp.float32) + b_ref[...]
    cols = jax.lax.broadcasted_iota(jnp.int32, s.shape, 1)
    m1 = jnp.max(s, axis=-1, keepdims=True)
    i1 = jnp.min(jnp.where(s == m1, cols, E), axis=-1, keepdims=True)
    s2 = jnp.where(cols == i1, -jnp.inf, s)
    m2 = jnp.max(s2, axis=-1, keepdims=True)
    i2 = jnp.min(jnp.where(s2 == m2, cols, E), axis=-1, keepdims=True)
    ex = jnp.exp(m2 - m1)
    w1 = 1.0 / (1.0 + ex)
    w2 = 1.0 - w1
    meta_ref[...] = jnp.concatenate(
        [w1, w2, i1.astype(jnp.float32), i2.astype(jnp.float32),
         jnp.zeros_like(s[:, :4])], axis=1)


def _gate(x, gw, gb):
    bs = 256
    return pl.pallas_call(
        _gate_kernel,
        grid=(S // bs,),
        in_specs=[
            pl.BlockSpec((bs, D), lambda i: (i, 0)),
            pl.BlockSpec((D, E), lambda i: (0, 0)),
            pl.BlockSpec((1, E), lambda i: (0, 0)),
        ],
        out_specs=pl.BlockSpec((bs, E), lambda i: (i, 0)),
        out_shape=jax.ShapeDtypeStruct((S, E), jnp.float32),
    )(x, gw, gb.reshape(1, E))


_BM = 128
_PT = 2 * S + E * _BM
_NMB = _PT // _BM

def _expert_kernel(be_ref, xs_ref, ws_ref, w1_ref, b1_ref, w2_ref, b2_ref,
                   o_ref):
    e = be_ref[pl.program_id(0)]
    h = (jnp.dot(xs_ref[...], w1_ref[0],
                 preferred_element_type=jnp.float32) + b1_ref[0])
    gelu = 0.5 * h * (1.0 + jax.lax.erf(h * (1.0 / math.sqrt(2.0))))
    silu = h * jax.nn.sigmoid(h)
    h = jnp.where(e % 2 == 0, gelu, silu)
    part = jnp.dot(h, w2_ref[0], preferred_element_type=jnp.float32)
    o_ref[...] = ws_ref[...] * (part + b2_ref[0])


def _expert_ffn(xs, ws, be, W1, b1, W2, b2):
    spec = pltpu.PrefetchScalarGridSpec(
        num_scalar_prefetch=1,
        grid=(_NMB,),
        in_specs=[
            pl.BlockSpec((_BM, D), lambda mb, be: (mb, 0)),
            pl.BlockSpec((_BM, 1), lambda mb, be: (mb, 0)),
            pl.BlockSpec((1, D, F), lambda mb, be: (be[mb], 0, 0)),
            pl.BlockSpec((1, 1, F), lambda mb, be: (be[mb], 0, 0)),
            pl.BlockSpec((1, F, D), lambda mb, be: (be[mb], 0, 0)),
            pl.BlockSpec((1, 1, D), lambda mb, be: (be[mb], 0, 0)),
        ],
        out_specs=pl.BlockSpec((_BM, D), lambda mb, be: (mb, 0)),
    )
    return pl.pallas_call(
        _expert_kernel,
        grid_spec=spec,
        out_shape=jax.ShapeDtypeStruct((_PT, D), jnp.float32),
    )(be, xs, ws, W1, b1.reshape(E, 1, F), W2, b2.reshape(E, 1, D))


def _combine_ln_kernel(ya_ref, yb_ref, x_ref, g_ref, b_ref, o_ref):
    y = ya_ref[...] + yb_ref[...] + x_ref[...]
    mu = jnp.mean(y, axis=-1, keepdims=True)
    yc = y - mu
    var = jnp.mean(yc * yc, axis=-1, keepdims=True)
    o_ref[...] = yc * jax.lax.rsqrt(var + 1e-5) * g_ref[...] + b_ref[...]


def _combine_ln(ypair, x, lnp):
    bs = 256
    spec = pl.BlockSpec((bs, D), lambda i: (i, 0))
    return pl.pallas_call(
        _combine_ln_kernel,
        grid=(S // bs,),
        in_specs=[
            spec,
            pl.BlockSpec((bs, D), lambda i: (i + S // bs, 0)),
            spec,
            pl.BlockSpec((1, D), lambda i: (0, 0)),
            pl.BlockSpec((1, D), lambda i: (0, 0)),
        ],
        out_specs=spec,
        out_shape=jax.ShapeDtypeStruct((S, D), jnp.float32),
    )(ypair, ypair, x, lnp['g'].reshape(1, D), lnp['b'].reshape(1, D))


def _moe_ln(p, x, lnp):
    meta = _gate(x, p['gate_W'], p['gate_b'])
    w1, w2 = meta[:, 0], meta[:, 1]
    e1 = meta[:, 2].astype(jnp.int32)
    e2 = meta[:, 3].astype(jnp.int32)

    ef = jnp.concatenate([e1, e2])
    wf = jnp.concatenate([w1, w2])
    tf = jnp.concatenate([jnp.arange(S, dtype=jnp.int32)] * 2)
    order = jnp.argsort(ef)
    ef_s, tf_s, wf_s = ef[order], tf[order], wf[order]
    counts = jnp.bincount(ef, length=E)
    starts = jnp.cumsum(counts) - counts
    pcounts = ((counts + _BM - 1) // _BM) * _BM
    pcsum = jnp.cumsum(pcounts)
    pstarts = pcsum - pcounts
    rank = jnp.arange(2 * S, dtype=jnp.int32) - starts[ef_s]
    pos_s = (pstarts[ef_s] + rank).astype(jnp.int32)
    src_row = jnp.zeros((_PT,), jnp.int32).at[pos_s].set(tf_s)
    ws = jnp.zeros((_PT,), jnp.float32).at[pos_s].set(wf_s)
    mbs = jnp.arange(_NMB, dtype=jnp.int32) * _BM
    be = jnp.clip(jnp.sum(pcsum[None, :] <= mbs[:, None], axis=1),
                  0, E - 1).astype(jnp.int32)
    posf = jnp.zeros((2 * S,), jnp.int32).at[order].set(pos_s)
    p1, p2 = posf[:S], posf[S:]

    xs = _sc_gather(x, src_row)
    ys = _expert_ffn(xs, ws.reshape(_PT, 1), be, p['W1'], p['b1'],
                     p['W2'], p['b2'])
    ypair = _sc_gather(ys, jnp.concatenate([p1, p2]))
    return _combine_ln(ypair, x, lnp)


def _block(p, x, enc_out=None):
    x = _add_ln(x, _mha(p['sa'], x, x, x), p['ln1'])
    if enc_out is not None:
        x = _add_ln(x, _mha(p['ca'], x, enc_out, enc_out), p['ln2'])
    return _moe_ln(p['moe'], x, p['ln3'])


def kernel(src, tgt, params):
    src = src.reshape(-1).astype(jnp.int32)
    tgt = tgt.reshape(-1).astype(jnp.int32)
    emb = params['embedding']
    c, a, b = _rope_coefs()
    se, te = _embed_rope2(src, tgt, emb, c, a, b)
    se = _block(params['enc'][0], se)
    te = _block(params['dec'][0], te, enc_out=se)
    logits = _matmul(te, params['out_W'], params['out_b'], 640)
    return logits.reshape(1, S, -1)
